# Initial kernel scaffold; baseline (speedup 1.0000x reference)
import jax
import jax.numpy as jnp
from jax.experimental import pallas as pl


def kernel(x, edge_index, W_l0, W_r0, b0, W_l1, W_r1, b1, W_fc, b_fc):
    raise NotImplementedError("write your pallas kernel here")


# speedup vs baseline: 1.7069x; 1.7069x over previous
import functools

import jax
import jax.numpy as jnp
from jax import lax
from jax.experimental import pallas as pl
from jax.experimental.pallas import tpu as pltpu
from jax.experimental.pallas import tpu_sc as plsc

_N = 10000
_E = 160000
_D = 256
_H = 256
_C = 64

_NT = 32
_OWN = 320
_NPAD = _NT * _OWN
_DUMMY = _OWN
_KS = 1280
_NCHUNK = _E // _KS
_K = 128
_QCAP = _E + _K
_QV = _KS + 256

_SC_PARAMS = pltpu.CompilerParams(needs_layout_passes=False)


def _mesh():
    return plsc.VectorSubcoreMesh(core_axis_name="c", subcore_axis_name="s")


@functools.partial(
    pl.kernel,
    out_type=[jax.ShapeDtypeStruct((_NT * _QCAP,), jnp.int32),
              jax.ShapeDtypeStruct((_NT * 128,), jnp.int32)],
    mesh=_mesh(),
    compiler_params=_SC_PARAMS,
    scratch_types=[
        pltpu.VMEM((_KS,), jnp.int32),
        pltpu.VMEM((_KS,), jnp.int32),
        pltpu.VMEM((_QV,), jnp.int32),
        pltpu.VMEM((128,), jnp.int32),
    ],
)
def _scan(src_hbm, dst_hbm, q_hbm, cnt_hbm, sstage, dstage, q_v, cv):
    c = lax.axis_index("c")
    s = lax.axis_index("s")
    w = c * 16 + s
    lo = w * _OWN

    def chunk(t, carry):
        cnt, qtot = carry
        e0 = pl.multiple_of(t * _KS, 128)
        pltpu.sync_copy(src_hbm.at[pl.ds(e0, _KS)], sstage)
        pltpu.sync_copy(dst_hbm.at[pl.ds(e0, _KS)], dstage)

        def vec(i, cnt):
            sv = sstage[pl.ds(i * 16, 16)]
            dv = dstage[pl.ds(i * 16, 16)]
            rel = dv - lo
            own = (rel >= 0) & (rel < _OWN)
            pk = (sv << 9) | jnp.where(own, rel, _DUMMY)
            key = jnp.where(own, 0, 1)
            _, vv = plsc.sort_key_val(key, pk)
            q_v[pl.ds(cnt, 16)] = vv
            return cnt + jnp.sum(jnp.where(own, 1, 0))

        cnt = lax.fori_loop(0, _KS // 16, vec, cnt)

        nb = cnt // _K

        def flush(b, carry2):
            o = pl.multiple_of(w * _QCAP + qtot + b * _K, 128)
            pltpu.sync_copy(q_v.at[pl.ds(b * _K, _K)], q_hbm.at[pl.ds(o, _K)])
            return carry2

        lax.fori_loop(0, nb, flush, 0)
        for j in range(_K // 16):
            q_v[pl.ds(j * 16, 16)] = q_v[pl.ds(nb * _K + j * 16, 16)]
        return cnt - nb * _K, qtot + nb * _K

    cnt, qtot = lax.fori_loop(0, _NCHUNK, chunk, (0, 0))

    pad = jnp.full((16,), _DUMMY, jnp.int32)
    for j in range(_K // 16):
        q_v[pl.ds(cnt + j * 16, 16)] = pad
    o = pl.multiple_of(w * _QCAP + qtot, 128)
    pltpu.sync_copy(q_v.at[pl.ds(0, _K)], q_hbm.at[pl.ds(o, _K)])
    qtot = qtot + _K

    for j in range(8):
        cv[pl.ds(j * 16, 16)] = jnp.zeros((16,), jnp.int32) + qtot
    pltpu.sync_copy(cv, cnt_hbm.at[pl.ds(pl.multiple_of(w * 128, 128), 128)])


def _make_aggregate(compute_deg: bool):
    out_types = [jax.ShapeDtypeStruct((_NPAD, _D), jnp.float32)]
    scratch = [
        pltpu.VMEM((_K,), jnp.int32),
        pltpu.VMEM((_K,), jnp.int32),
        pltpu.VMEM((_K + 16,), jnp.int32),
        pltpu.VMEM((_K, _D), jnp.float32),
        pltpu.VMEM((_OWN + 8, _D), jnp.float32),
        pltpu.VMEM((128,), jnp.int32),
        pltpu.SemaphoreType.DMA,
    ]
    if compute_deg:
        out_types.append(jax.ShapeDtypeStruct((_NPAD * 16,), jnp.float32))
        scratch.append(pltpu.VMEM(((_OWN + 8) * 16,), jnp.float32))

    @functools.partial(
        pl.kernel,
        out_type=out_types,
        mesh=_mesh(),
        compiler_params=_SC_PARAMS,
        scratch_types=scratch,
    )
    def agg(table_hbm, q_hbm, cnt_hbm, zd_hbm, z16_hbm, *refs):
        if compute_deg:
            (out_hbm, deg_hbm, bl_v, gidx_v, lidx_v, rows_v, acc_v, cv, sem,
             deg_v) = refs
        else:
            out_hbm, bl_v, gidx_v, lidx_v, rows_v, acc_v, cv, sem = refs
        c = lax.axis_index("c")
        s = lax.axis_index("s")
        w = c * 16 + s

        pltpu.sync_copy(zd_hbm, acc_v)
        if compute_deg:
            pltpu.sync_copy(z16_hbm, deg_v)
        pltpu.sync_copy(cnt_hbm.at[pl.ds(pl.multiple_of(w * 128, 128), 128)],
                        cv)
        qn = cv[pl.ds(0, 16)][0]
        one = jnp.ones((16,), jnp.float32)

        def batch(b, carry):
            o = pl.multiple_of(w * _QCAP + b * _K, 128)
            pltpu.sync_copy(q_hbm.at[pl.ds(o, _K)], bl_v)
            for j in range(_K // 16):
                v = bl_v[pl.ds(j * 16, 16)]
                gidx_v[pl.ds(j * 16, 16)] = v >> 9
                lidx_v[pl.ds(j * 16, 16)] = v & 511
            pltpu.async_copy(table_hbm.at[gidx_v], rows_v, sem).wait()

            def row(r, carry2):
                ld = lidx_v[pl.ds(r, 16)][0]
                for j in range(_D // 16):
                    plsc.addupdate(acc_v.at[ld, pl.ds(j * 16, 16)],
                                   rows_v[r, pl.ds(j * 16, 16)])
                if compute_deg:
                    plsc.addupdate(deg_v.at[pl.ds(ld * 16, 16)], one)
                return carry2

            lax.fori_loop(0, _K, row, 0)
            return carry

        lax.fori_loop(0, qn // _K, batch, 0)

        pltpu.sync_copy(acc_v.at[pl.ds(0, _OWN)],
                        out_hbm.at[pl.ds(pl.multiple_of(w * _OWN, 8), _OWN)])
        if compute_deg:
            pltpu.sync_copy(deg_v.at[pl.ds(0, _OWN * 16)],
                            deg_hbm.at[pl.ds(pl.multiple_of(w * _OWN * 16,
                                                            128),
                                             _OWN * 16)])

    return agg


_aggregate_deg = _make_aggregate(True)
_aggregate_nodeg = _make_aggregate(False)

_BR = 1000


def _sage_layer_tc(aggsum, deg16, h_in, Wl, Wr, b, relu: bool):
    def body(agg_ref, deg_ref, h_ref, wl_ref, wr_ref, b_ref, o_ref):
        deg = jnp.maximum(deg_ref[:, 0:1], 1.0)
        agg = agg_ref[...] / deg
        o = (jnp.dot(agg, wl_ref[...], preferred_element_type=jnp.float32)
             + jnp.dot(h_ref[...], wr_ref[...],
                       preferred_element_type=jnp.float32)
             + b_ref[...])
        if relu:
            o = jnp.maximum(o, 0.0)
        o_ref[...] = o

    return pl.pallas_call(
        body,
        grid=(_N // _BR,),
        in_specs=[
            pl.BlockSpec((_BR, _D), lambda i: (i, 0)),
            pl.BlockSpec((_BR, 16), lambda i: (i, 0)),
            pl.BlockSpec((_BR, _D), lambda i: (i, 0)),
            pl.BlockSpec((_D, _H), lambda i: (0, 0)),
            pl.BlockSpec((_D, _H), lambda i: (0, 0)),
            pl.BlockSpec((1, _H), lambda i: (0, 0)),
        ],
        out_specs=pl.BlockSpec((_BR, _H), lambda i: (i, 0)),
        out_shape=jax.ShapeDtypeStruct((_N, _H), jnp.float32),
    )(aggsum, deg16, h_in, Wl, Wr, b.reshape(1, _H))


def _final_tc(aggsum, deg16, h_in, Wl, Wr, b, Wfc, bfc):
    def body(agg_ref, deg_ref, h_ref, wl_ref, wr_ref, b_ref, wfc_ref,
             bfc_ref, o_ref):
        deg = jnp.maximum(deg_ref[:, 0:1], 1.0)
        agg = agg_ref[...] / deg
        h2 = (jnp.dot(agg, wl_ref[...], preferred_element_type=jnp.float32)
              + jnp.dot(h_ref[...], wr_ref[...],
                        preferred_element_type=jnp.float32)
              + b_ref[...])
        z = (jnp.dot(h2, wfc_ref[...], preferred_element_type=jnp.float32)
             + bfc_ref[...])
        m = jnp.max(z, axis=-1, keepdims=True)
        e = jnp.exp(z - m)
        o_ref[...] = z - m - jnp.log(jnp.sum(e, axis=-1, keepdims=True))

    return pl.pallas_call(
        body,
        grid=(_N // _BR,),
        in_specs=[
            pl.BlockSpec((_BR, _D), lambda i: (i, 0)),
            pl.BlockSpec((_BR, 16), lambda i: (i, 0)),
            pl.BlockSpec((_BR, _H), lambda i: (i, 0)),
            pl.BlockSpec((_H, _H), lambda i: (0, 0)),
            pl.BlockSpec((_H, _H), lambda i: (0, 0)),
            pl.BlockSpec((1, _H), lambda i: (0, 0)),
            pl.BlockSpec((_H, _C), lambda i: (0, 0)),
            pl.BlockSpec((1, _C), lambda i: (0, 0)),
        ],
        out_specs=pl.BlockSpec((_BR, _C), lambda i: (i, 0)),
        out_shape=jax.ShapeDtypeStruct((_N, _C), jnp.float32),
    )(aggsum, deg16, h_in, Wl, Wr, b.reshape(1, _H), Wfc, bfc.reshape(1, _C))


def kernel(x, edge_index, W_l0, W_r0, b0, W_l1, W_r1, b1, W_fc, b_fc):
    src = edge_index[0]
    dst = edge_index[1]
    zd = jnp.zeros((_OWN + 8, _D), jnp.float32)
    z16 = jnp.zeros(((_OWN + 8) * 16,), jnp.float32)

    q, counts = _scan(src, dst)
    agg0, deg = _aggregate_deg(x, q, counts, zd, z16)
    deg = deg.reshape(_NPAD, 16)
    h = _sage_layer_tc(agg0, deg, x, W_l0, W_r0, b0, relu=True)
    (agg1,) = _aggregate_nodeg(h, q, counts, zd, z16)
    return _final_tc(agg1, deg, h, W_l1, W_r1, b1, W_fc, b_fc)


# speedup vs baseline: 1.8925x; 1.1087x over previous
import functools

import jax
import jax.numpy as jnp
from jax import lax
from jax.experimental import pallas as pl
from jax.experimental.pallas import tpu as pltpu
from jax.experimental.pallas import tpu_sc as plsc

_N = 10000
_E = 160000
_D = 256
_H = 256
_C = 64

_NT = 32
_OWN = 320
_NPAD = _NT * _OWN
_DUMMY = _OWN
_KS = 1280
_NCHUNK = _E // _KS
_K = 128
_GB = 64
_SQ = 1024
_NBLK = _SQ // _GB
_QCAP = 157 * _SQ
_QV = _KS + 256

_SC_PARAMS = pltpu.CompilerParams(needs_layout_passes=False)


def _mesh():
    return plsc.VectorSubcoreMesh(core_axis_name="c", subcore_axis_name="s")


@functools.partial(
    pl.kernel,
    out_type=[jax.ShapeDtypeStruct((_NT * _QCAP,), jnp.int32),
              jax.ShapeDtypeStruct((_NT * 128,), jnp.int32)],
    mesh=_mesh(),
    compiler_params=_SC_PARAMS,
    scratch_types=[
        pltpu.VMEM((_KS,), jnp.int32),
        pltpu.VMEM((_KS,), jnp.int32),
        pltpu.VMEM((_QV,), jnp.int32),
        pltpu.VMEM((128,), jnp.int32),
    ],
)
def _scan(src_hbm, dst_hbm, q_hbm, cnt_hbm, sstage, dstage, q_v, cv):
    c = lax.axis_index("c")
    s = lax.axis_index("s")
    w = c * 16 + s
    lo = w * _OWN

    def chunk(t, carry):
        cnt, qtot = carry
        e0 = pl.multiple_of(t * _KS, 128)
        pltpu.sync_copy(src_hbm.at[pl.ds(e0, _KS)], sstage)
        pltpu.sync_copy(dst_hbm.at[pl.ds(e0, _KS)], dstage)

        def vec(i, cnt):
            sv = sstage[pl.ds(i * 16, 16)]
            dv = dstage[pl.ds(i * 16, 16)]
            rel = dv - lo
            own = (rel >= 0) & (rel < _OWN)
            pk = (sv << 9) | jnp.where(own, rel, _DUMMY)
            key = jnp.where(own, 0, 1)
            _, vv = plsc.sort_key_val(key, pk)
            q_v[pl.ds(cnt, 16)] = vv
            return cnt + jnp.sum(jnp.where(own, 1, 0))

        cnt = lax.fori_loop(0, _KS // 16, vec, cnt)

        nb = cnt // _K

        def flush(b, carry2):
            o = pl.multiple_of(w * _QCAP + qtot + b * _K, 128)
            pltpu.sync_copy(q_v.at[pl.ds(b * _K, _K)], q_hbm.at[pl.ds(o, _K)])
            return carry2

        lax.fori_loop(0, nb, flush, 0)
        for j in range(_K // 16):
            q_v[pl.ds(j * 16, 16)] = q_v[pl.ds(nb * _K + j * 16, 16)]
        return cnt - nb * _K, qtot + nb * _K

    cnt, qtot = lax.fori_loop(0, _NCHUNK, chunk, (0, 0))

    pad = jnp.full((16,), _DUMMY, jnp.int32)
    for j in range(_K // 16):
        q_v[pl.ds(cnt + j * 16, 16)] = pad
    o = pl.multiple_of(w * _QCAP + qtot, 128)
    pltpu.sync_copy(q_v.at[pl.ds(0, _K)], q_hbm.at[pl.ds(o, _K)])
    qtot = qtot + _K

    for j in range(8):
        cv[pl.ds(j * 16, 16)] = jnp.zeros((16,), jnp.int32) + qtot
    pltpu.sync_copy(cv, cnt_hbm.at[pl.ds(pl.multiple_of(w * 128, 128), 128)])


def _make_aggregate(compute_deg: bool):
    out_types = [jax.ShapeDtypeStruct((_NPAD, _D), jnp.float32)]
    scratch = [
        pltpu.VMEM((_SQ,), jnp.int32),
        pltpu.VMEM((_NBLK, _GB), jnp.int32),
        pltpu.VMEM((_NBLK, _GB + 16), jnp.int32),
        pltpu.VMEM((_GB, _D), jnp.float32),
        pltpu.VMEM((_GB, _D), jnp.float32),
        pltpu.VMEM((_OWN + 8, _D), jnp.float32),
        pltpu.VMEM((128,), jnp.int32),
        pltpu.SemaphoreType.DMA,
        pltpu.SemaphoreType.DMA,
    ]
    if compute_deg:
        out_types.append(jax.ShapeDtypeStruct((_NPAD * 16,), jnp.float32))
        scratch.append(pltpu.VMEM(((_OWN + 8) * 16,), jnp.float32))

    @functools.partial(
        pl.kernel,
        out_type=out_types,
        mesh=_mesh(),
        compiler_params=_SC_PARAMS,
        scratch_types=scratch,
    )
    def agg(table_hbm, q_hbm, cnt_hbm, zd_hbm, z16_hbm, *refs):
        if compute_deg:
            (out_hbm, deg_hbm, sq_v, gidx_v, lidx_v, rows0, rows1, acc_v, cv,
             sem0, sem1, deg_v) = refs
        else:
            (out_hbm, sq_v, gidx_v, lidx_v, rows0, rows1, acc_v, cv,
             sem0, sem1) = refs
        rows = (rows0, rows1)
        sems = (sem0, sem1)
        c = lax.axis_index("c")
        s = lax.axis_index("s")
        w = c * 16 + s

        pltpu.sync_copy(zd_hbm, acc_v)
        if compute_deg:
            pltpu.sync_copy(z16_hbm, deg_v)
        pltpu.sync_copy(cnt_hbm.at[pl.ds(pl.multiple_of(w * 128, 128), 128)],
                        cv)
        qn = cv[pl.ds(0, 16)][0]
        one = jnp.ones((16,), jnp.float32)

        def prep_fire(k):
            for j in range(_GB // 16):
                v = sq_v[pl.ds(k * _GB + j * 16, 16)]
                gidx_v[k, pl.ds(j * 16, 16)] = v >> 9
                lidx_v[k, pl.ds(j * 16, 16)] = v & 511
            pltpu.async_copy(table_hbm.at[gidx_v.at[k]], rows[k % 2],
                             sems[k % 2])

        def acc_block(k):
            pltpu.make_async_copy(table_hbm.at[gidx_v.at[k]], rows[k % 2],
                                  sems[k % 2]).wait()

            def row(r, carry2):
                ld = lidx_v[k, pl.ds(r, 16)][0]
                for j in range(_D // 16):
                    plsc.addupdate(acc_v.at[ld, pl.ds(j * 16, 16)],
                                   rows[k % 2][r, pl.ds(j * 16, 16)])
                if compute_deg:
                    plsc.addupdate(deg_v.at[pl.ds(ld * 16, 16)], one)
                return carry2

            lax.fori_loop(0, _GB, row, 0)

        def superchunk(base, nblk):
            o = pl.multiple_of(w * _QCAP + base, 128)
            pltpu.sync_copy(q_hbm.at[pl.ds(o, _SQ)], sq_v)
            for k in range(_NBLK):
                if isinstance(nblk, int):
                    prep_fire(k)
                    if k >= 1:
                        acc_block(k - 1)
                else:
                    @pl.when(k < nblk)
                    def _():
                        prep_fire(k)
                    if k >= 1:
                        @pl.when(k - 1 < nblk)
                        def _():
                            acc_block(k - 1)
            if isinstance(nblk, int):
                acc_block(_NBLK - 1)
            else:
                @pl.when(nblk >= _NBLK)
                def _():
                    acc_block(_NBLK - 1)

        nsq = qn // _SQ

        def full(sc, carry):
            superchunk(sc * _SQ, _NBLK)
            return carry

        lax.fori_loop(0, nsq, full, 0)
        superchunk(nsq * _SQ, (qn - nsq * _SQ) // _GB)

        pltpu.sync_copy(acc_v.at[pl.ds(0, _OWN)],
                        out_hbm.at[pl.ds(pl.multiple_of(w * _OWN, 8), _OWN)])
        if compute_deg:
            pltpu.sync_copy(deg_v.at[pl.ds(0, _OWN * 16)],
                            deg_hbm.at[pl.ds(pl.multiple_of(w * _OWN * 16,
                                                            128),
                                             _OWN * 16)])

    return agg


_aggregate_deg = _make_aggregate(True)
_aggregate_nodeg = _make_aggregate(False)

_BR = 1000


def _sage_layer_tc(aggsum, deg16, h_in, Wl, Wr, b, relu: bool):
    def body(agg_ref, deg_ref, h_ref, wl_ref, wr_ref, b_ref, o_ref):
        deg = jnp.maximum(deg_ref[:, 0:1], 1.0)
        agg = agg_ref[...] / deg
        o = (jnp.dot(agg, wl_ref[...], preferred_element_type=jnp.float32)
             + jnp.dot(h_ref[...], wr_ref[...],
                       preferred_element_type=jnp.float32)
             + b_ref[...])
        if relu:
            o = jnp.maximum(o, 0.0)
        o_ref[...] = o

    return pl.pallas_call(
        body,
        grid=(_N // _BR,),
        in_specs=[
            pl.BlockSpec((_BR, _D), lambda i: (i, 0)),
            pl.BlockSpec((_BR, 16), lambda i: (i, 0)),
            pl.BlockSpec((_BR, _D), lambda i: (i, 0)),
            pl.BlockSpec((_D, _H), lambda i: (0, 0)),
            pl.BlockSpec((_D, _H), lambda i: (0, 0)),
            pl.BlockSpec((1, _H), lambda i: (0, 0)),
        ],
        out_specs=pl.BlockSpec((_BR, _H), lambda i: (i, 0)),
        out_shape=jax.ShapeDtypeStruct((_N, _H), jnp.float32),
    )(aggsum, deg16, h_in, Wl, Wr, b.reshape(1, _H))


def _final_tc(aggsum, deg16, h_in, Wl, Wr, b, Wfc, bfc):
    def body(agg_ref, deg_ref, h_ref, wl_ref, wr_ref, b_ref, wfc_ref,
             bfc_ref, o_ref):
        deg = jnp.maximum(deg_ref[:, 0:1], 1.0)
        agg = agg_ref[...] / deg
        h2 = (jnp.dot(agg, wl_ref[...], preferred_element_type=jnp.float32)
              + jnp.dot(h_ref[...], wr_ref[...],
                        preferred_element_type=jnp.float32)
              + b_ref[...])
        z = (jnp.dot(h2, wfc_ref[...], preferred_element_type=jnp.float32)
             + bfc_ref[...])
        m = jnp.max(z, axis=-1, keepdims=True)
        e = jnp.exp(z - m)
        o_ref[...] = z - m - jnp.log(jnp.sum(e, axis=-1, keepdims=True))

    return pl.pallas_call(
        body,
        grid=(_N // _BR,),
        in_specs=[
            pl.BlockSpec((_BR, _D), lambda i: (i, 0)),
            pl.BlockSpec((_BR, 16), lambda i: (i, 0)),
            pl.BlockSpec((_BR, _H), lambda i: (i, 0)),
            pl.BlockSpec((_H, _H), lambda i: (0, 0)),
            pl.BlockSpec((_H, _H), lambda i: (0, 0)),
            pl.BlockSpec((1, _H), lambda i: (0, 0)),
            pl.BlockSpec((_H, _C), lambda i: (0, 0)),
            pl.BlockSpec((1, _C), lambda i: (0, 0)),
        ],
        out_specs=pl.BlockSpec((_BR, _C), lambda i: (i, 0)),
        out_shape=jax.ShapeDtypeStruct((_N, _C), jnp.float32),
    )(aggsum, deg16, h_in, Wl, Wr, b.reshape(1, _H), Wfc, bfc.reshape(1, _C))


def kernel(x, edge_index, W_l0, W_r0, b0, W_l1, W_r1, b1, W_fc, b_fc):
    src = edge_index[0]
    dst = edge_index[1]
    zd = jnp.zeros((_OWN + 8, _D), jnp.float32)
    z16 = jnp.zeros(((_OWN + 8) * 16,), jnp.float32)

    q, counts = _scan(src, dst)
    agg0, deg = _aggregate_deg(x, q, counts, zd, z16)
    deg = deg.reshape(_NPAD, 16)
    h = _sage_layer_tc(agg0, deg, x, W_l0, W_r0, b0, relu=True)
    (agg1,) = _aggregate_nodeg(h, q, counts, zd, z16)
    return _final_tc(agg1, deg, h, W_l1, W_r1, b1, W_fc, b_fc)


# speedup vs baseline: 1.9178x; 1.0134x over previous
import functools

import jax
import jax.numpy as jnp
from jax import lax
from jax.experimental import pallas as pl
from jax.experimental.pallas import tpu as pltpu
from jax.experimental.pallas import tpu_sc as plsc

_N = 10000
_E = 160000
_D = 256
_H = 256
_C = 64

_NT = 32
_OWN = 320
_NPAD = _NT * _OWN
_DUMMY = _OWN
_KS = 1280
_NCHUNK = _E // _KS
_K = 128
_GB = 64
_SQ = 256
_NBLK = _SQ // _GB
_QCAP = 626 * _SQ
_QV = _KS + 256

_SC_PARAMS = pltpu.CompilerParams(needs_layout_passes=False)


def _mesh():
    return plsc.VectorSubcoreMesh(core_axis_name="c", subcore_axis_name="s")


@functools.partial(
    pl.kernel,
    out_type=[jax.ShapeDtypeStruct((_NT * _QCAP,), jnp.int32),
              jax.ShapeDtypeStruct((_NT * 128,), jnp.int32)],
    mesh=_mesh(),
    compiler_params=_SC_PARAMS,
    scratch_types=[
        pltpu.VMEM((_KS,), jnp.int32),
        pltpu.VMEM((_KS,), jnp.int32),
        pltpu.VMEM((_QV,), jnp.int32),
        pltpu.VMEM((128,), jnp.int32),
    ],
)
def _scan(src_hbm, dst_hbm, q_hbm, cnt_hbm, sstage, dstage, q_v, cv):
    c = lax.axis_index("c")
    s = lax.axis_index("s")
    w = c * 16 + s
    lo = w * _OWN

    def chunk(t, carry):
        cnt, qtot = carry
        e0 = pl.multiple_of(t * _KS, 128)
        pltpu.sync_copy(src_hbm.at[pl.ds(e0, _KS)], sstage)
        pltpu.sync_copy(dst_hbm.at[pl.ds(e0, _KS)], dstage)

        def vec(i, cnt):
            sv = sstage[pl.ds(i * 16, 16)]
            dv = dstage[pl.ds(i * 16, 16)]
            rel = dv - lo
            own = rel.astype(jnp.uint32) < jnp.uint32(_OWN)
            pk = (sv << 9) | jnp.where(own, rel, _DUMMY)
            key = jnp.where(own, 0, 1)
            _, vv = plsc.sort_key_val(key, pk)
            q_v[pl.ds(cnt, 16)] = vv
            return cnt + plsc.all_reduce_population_count(own)[0]

        cnt = lax.fori_loop(0, _KS // 16, vec, cnt)

        nb = cnt // _K

        def flush(b, carry2):
            o = pl.multiple_of(w * _QCAP + qtot + b * _K, 128)
            pltpu.sync_copy(q_v.at[pl.ds(b * _K, _K)], q_hbm.at[pl.ds(o, _K)])
            return carry2

        lax.fori_loop(0, nb, flush, 0)
        for j in range(_K // 16):
            q_v[pl.ds(j * 16, 16)] = q_v[pl.ds(nb * _K + j * 16, 16)]
        return cnt - nb * _K, qtot + nb * _K

    cnt, qtot = lax.fori_loop(0, _NCHUNK, chunk, (0, 0))

    pad = jnp.full((16,), _DUMMY, jnp.int32)
    for j in range(_K // 16):
        q_v[pl.ds(cnt + j * 16, 16)] = pad
    o = pl.multiple_of(w * _QCAP + qtot, 128)
    pltpu.sync_copy(q_v.at[pl.ds(0, _K)], q_hbm.at[pl.ds(o, _K)])
    qtot = qtot + _K

    for j in range(8):
        cv[pl.ds(j * 16, 16)] = jnp.zeros((16,), jnp.int32) + qtot
    pltpu.sync_copy(cv, cnt_hbm.at[pl.ds(pl.multiple_of(w * 128, 128), 128)])


def _make_aggregate(compute_deg: bool):
    out_types = [jax.ShapeDtypeStruct((_NPAD, _D), jnp.float32)]
    scratch = [
        pltpu.VMEM((_SQ,), jnp.int32),
        pltpu.VMEM((_NBLK, _GB), jnp.int32),
        pltpu.VMEM((_NBLK, _GB + 16), jnp.int32),
        pltpu.VMEM((_GB, _D), jnp.float32),
        pltpu.VMEM((_GB, _D), jnp.float32),
        pltpu.VMEM((_OWN + 8, _D), jnp.float32),
        pltpu.VMEM((128,), jnp.int32),
        pltpu.SemaphoreType.DMA,
        pltpu.SemaphoreType.DMA,
    ]
    if compute_deg:
        out_types.append(jax.ShapeDtypeStruct((_NPAD * 16,), jnp.float32))
        scratch.append(pltpu.VMEM(((_OWN + 8) * 16,), jnp.float32))

    @functools.partial(
        pl.kernel,
        out_type=out_types,
        mesh=_mesh(),
        compiler_params=_SC_PARAMS,
        scratch_types=scratch,
    )
    def agg(table_hbm, q_hbm, cnt_hbm, zd_hbm, z16_hbm, *refs):
        if compute_deg:
            (out_hbm, deg_hbm, sq_v, gidx_v, lidx_v, rows0, rows1, acc_v, cv,
             sem0, sem1, deg_v) = refs
        else:
            (out_hbm, sq_v, gidx_v, lidx_v, rows0, rows1, acc_v, cv,
             sem0, sem1) = refs
        rows = (rows0, rows1)
        sems = (sem0, sem1)
        c = lax.axis_index("c")
        s = lax.axis_index("s")
        w = c * 16 + s

        pltpu.sync_copy(zd_hbm, acc_v)
        if compute_deg:
            pltpu.sync_copy(z16_hbm, deg_v)
        pltpu.sync_copy(cnt_hbm.at[pl.ds(pl.multiple_of(w * 128, 128), 128)],
                        cv)
        qn = cv[pl.ds(0, 16)][0]
        one = jnp.ones((16,), jnp.float32)

        def prep_fire(k):
            for j in range(_GB // 16):
                v = sq_v[pl.ds(k * _GB + j * 16, 16)]
                gidx_v[k, pl.ds(j * 16, 16)] = v >> 9
                lidx_v[k, pl.ds(j * 16, 16)] = v & 511
            pltpu.async_copy(table_hbm.at[gidx_v.at[k]], rows[k % 2],
                             sems[k % 2])

        def acc_block(k):
            pltpu.make_async_copy(table_hbm.at[gidx_v.at[k]], rows[k % 2],
                                  sems[k % 2]).wait()

            def row_group(g, carry2):
                lv = lidx_v[k, pl.ds(g * 16, 16)]
                for i in range(16):
                    ld = lv[i]
                    r = g * 16 + i
                    for j in range(_D // 16):
                        plsc.addupdate(acc_v.at[ld, pl.ds(j * 16, 16)],
                                       rows[k % 2][r, pl.ds(j * 16, 16)])
                    if compute_deg:
                        plsc.addupdate(deg_v.at[pl.ds(ld * 16, 16)], one)
                return carry2

            lax.fori_loop(0, _GB // 16, row_group, 0)

        def superchunk(base, nblk):
            o = pl.multiple_of(w * _QCAP + base, 128)
            pltpu.sync_copy(q_hbm.at[pl.ds(o, _SQ)], sq_v)
            for k in range(_NBLK):
                if isinstance(nblk, int):
                    prep_fire(k)
                    if k >= 1:
                        acc_block(k - 1)
                else:
                    @pl.when(k < nblk)
                    def _():
                        prep_fire(k)
                    if k >= 1:
                        @pl.when(k - 1 < nblk)
                        def _():
                            acc_block(k - 1)
            if isinstance(nblk, int):
                acc_block(_NBLK - 1)
            else:
                @pl.when(nblk >= _NBLK)
                def _():
                    acc_block(_NBLK - 1)

        nsq = qn // _SQ

        def full(sc, carry):
            superchunk(sc * _SQ, _NBLK)
            return carry

        lax.fori_loop(0, nsq, full, 0)
        superchunk(nsq * _SQ, (qn - nsq * _SQ) // _GB)

        pltpu.sync_copy(acc_v.at[pl.ds(0, _OWN)],
                        out_hbm.at[pl.ds(pl.multiple_of(w * _OWN, 8), _OWN)])
        if compute_deg:
            pltpu.sync_copy(deg_v.at[pl.ds(0, _OWN * 16)],
                            deg_hbm.at[pl.ds(pl.multiple_of(w * _OWN * 16,
                                                            128),
                                             _OWN * 16)])

    return agg


_aggregate_deg = _make_aggregate(True)
_aggregate_nodeg = _make_aggregate(False)

_BR = 1000


def _sage_layer_tc(aggsum, deg16, h_in, Wl, Wr, b, relu: bool):
    def body(agg_ref, deg_ref, h_ref, wl_ref, wr_ref, b_ref, o_ref):
        deg = jnp.maximum(deg_ref[:, 0:1], 1.0)
        agg = agg_ref[...] / deg
        o = (jnp.dot(agg, wl_ref[...], preferred_element_type=jnp.float32)
             + jnp.dot(h_ref[...], wr_ref[...],
                       preferred_element_type=jnp.float32)
             + b_ref[...])
        if relu:
            o = jnp.maximum(o, 0.0)
        o_ref[...] = o

    return pl.pallas_call(
        body,
        grid=(_N // _BR,),
        in_specs=[
            pl.BlockSpec((_BR, _D), lambda i: (i, 0)),
            pl.BlockSpec((_BR, 16), lambda i: (i, 0)),
            pl.BlockSpec((_BR, _D), lambda i: (i, 0)),
            pl.BlockSpec((_D, _H), lambda i: (0, 0)),
            pl.BlockSpec((_D, _H), lambda i: (0, 0)),
            pl.BlockSpec((1, _H), lambda i: (0, 0)),
        ],
        out_specs=pl.BlockSpec((_BR, _H), lambda i: (i, 0)),
        out_shape=jax.ShapeDtypeStruct((_N, _H), jnp.float32),
    )(aggsum, deg16, h_in, Wl, Wr, b.reshape(1, _H))


def _final_tc(aggsum, deg16, h_in, Wl, Wr, b, Wfc, bfc):
    def body(agg_ref, deg_ref, h_ref, wl_ref, wr_ref, b_ref, wfc_ref,
             bfc_ref, o_ref):
        deg = jnp.maximum(deg_ref[:, 0:1], 1.0)
        agg = agg_ref[...] / deg
        h2 = (jnp.dot(agg, wl_ref[...], preferred_element_type=jnp.float32)
              + jnp.dot(h_ref[...], wr_ref[...],
                        preferred_element_type=jnp.float32)
              + b_ref[...])
        z = (jnp.dot(h2, wfc_ref[...], preferred_element_type=jnp.float32)
             + bfc_ref[...])
        m = jnp.max(z, axis=-1, keepdims=True)
        e = jnp.exp(z - m)
        o_ref[...] = z - m - jnp.log(jnp.sum(e, axis=-1, keepdims=True))

    return pl.pallas_call(
        body,
        grid=(_N // _BR,),
        in_specs=[
            pl.BlockSpec((_BR, _D), lambda i: (i, 0)),
            pl.BlockSpec((_BR, 16), lambda i: (i, 0)),
            pl.BlockSpec((_BR, _H), lambda i: (i, 0)),
            pl.BlockSpec((_H, _H), lambda i: (0, 0)),
            pl.BlockSpec((_H, _H), lambda i: (0, 0)),
            pl.BlockSpec((1, _H), lambda i: (0, 0)),
            pl.BlockSpec((_H, _C), lambda i: (0, 0)),
            pl.BlockSpec((1, _C), lambda i: (0, 0)),
        ],
        out_specs=pl.BlockSpec((_BR, _C), lambda i: (i, 0)),
        out_shape=jax.ShapeDtypeStruct((_N, _C), jnp.float32),
    )(aggsum, deg16, h_in, Wl, Wr, b.reshape(1, _H), Wfc, bfc.reshape(1, _C))


def kernel(x, edge_index, W_l0, W_r0, b0, W_l1, W_r1, b1, W_fc, b_fc):
    src = edge_index[0]
    dst = edge_index[1]
    zd = jnp.zeros((_OWN + 8, _D), jnp.float32)
    z16 = jnp.zeros(((_OWN + 8) * 16,), jnp.float32)

    q, counts = _scan(src, dst)
    agg0, deg = _aggregate_deg(x, q, counts, zd, z16)
    deg = deg.reshape(_NPAD, 16)
    h = _sage_layer_tc(agg0, deg, x, W_l0, W_r0, b0, relu=True)
    (agg1,) = _aggregate_nodeg(h, q, counts, zd, z16)
    return _final_tc(agg1, deg, h, W_l1, W_r1, b1, W_fc, b_fc)


# speedup vs baseline: 2.6367x; 1.3748x over previous
import functools

import jax
import jax.numpy as jnp
from jax import lax
from jax.experimental import pallas as pl
from jax.experimental.pallas import tpu as pltpu
from jax.experimental.pallas import tpu_sc as plsc

_N = 10000
_E = 160000
_D = 256
_H = 256
_C = 64

_NT = 32
_OWN = 320
_NPAD = _NT * _OWN
_DUMMY = _OWN
_KS = 1280
_NCHUNK = _E // _KS
_K = 128
_GB = 64
_SQ = 256
_NBLK = _SQ // _GB
_QCAP = 626 * _SQ
_QV = _KS + 256

_SC_PARAMS = pltpu.CompilerParams(needs_layout_passes=False)


def _mesh():
    return plsc.VectorSubcoreMesh(core_axis_name="c", subcore_axis_name="s")


@functools.partial(
    pl.kernel,
    out_type=[jax.ShapeDtypeStruct((_NT * _QCAP,), jnp.int32),
              jax.ShapeDtypeStruct((_NT * 128,), jnp.int32)],
    mesh=_mesh(),
    compiler_params=_SC_PARAMS,
    scratch_types=[
        pltpu.VMEM((_KS,), jnp.int32),
        pltpu.VMEM((_KS,), jnp.int32),
        pltpu.VMEM((_QV,), jnp.int32),
        pltpu.VMEM((128,), jnp.int32),
    ],
)
def _scan(src_hbm, dst_hbm, q_hbm, cnt_hbm, sstage, dstage, q_v, cv):
    c = lax.axis_index("c")
    s = lax.axis_index("s")
    w = c * 16 + s
    lo = w * _OWN

    def chunk(t, carry):
        cnt, qtot = carry
        e0 = pl.multiple_of(t * _KS, 128)
        pltpu.sync_copy(src_hbm.at[pl.ds(e0, _KS)], sstage)
        pltpu.sync_copy(dst_hbm.at[pl.ds(e0, _KS)], dstage)

        def vec(i, cnt):
            sv = sstage[pl.ds(i * 16, 16)]
            dv = dstage[pl.ds(i * 16, 16)]
            rel = dv - lo
            own = rel.astype(jnp.uint32) < jnp.uint32(_OWN)
            pk = (sv << 9) | jnp.where(own, rel, _DUMMY)
            key = jnp.where(own, 0, 1)
            _, vv = plsc.sort_key_val(key, pk)
            q_v[pl.ds(cnt, 16)] = vv
            return cnt + plsc.all_reduce_population_count(own)[0]

        cnt = lax.fori_loop(0, _KS // 16, vec, cnt)

        nb = cnt // _K

        def flush(b, carry2):
            o = pl.multiple_of(w * _QCAP + qtot + b * _K, 128)
            pltpu.sync_copy(q_v.at[pl.ds(b * _K, _K)], q_hbm.at[pl.ds(o, _K)])
            return carry2

        lax.fori_loop(0, nb, flush, 0)
        for j in range(_K // 16):
            q_v[pl.ds(j * 16, 16)] = q_v[pl.ds(nb * _K + j * 16, 16)]
        return cnt - nb * _K, qtot + nb * _K

    cnt, qtot = lax.fori_loop(0, _NCHUNK, chunk, (0, 0))

    pad = jnp.full((16,), _DUMMY, jnp.int32)
    for j in range(_K // 16):
        q_v[pl.ds(cnt + j * 16, 16)] = pad
    o = pl.multiple_of(w * _QCAP + qtot, 128)
    pltpu.sync_copy(q_v.at[pl.ds(0, _K)], q_hbm.at[pl.ds(o, _K)])
    qtot = qtot + _K

    for j in range(8):
        cv[pl.ds(j * 16, 16)] = jnp.zeros((16,), jnp.int32) + qtot
    pltpu.sync_copy(cv, cnt_hbm.at[pl.ds(pl.multiple_of(w * 128, 128), 128)])


def _make_aggregate(compute_deg: bool):
    out_types = [jax.ShapeDtypeStruct((_NPAD, _D), jnp.float32)]
    scratch = [
        pltpu.VMEM((_SQ,), jnp.int32),
        pltpu.VMEM((_NBLK, _GB), jnp.int32),
        pltpu.VMEM((_NBLK, _GB + 16), jnp.int32),
        pltpu.VMEM((_GB, _D), jnp.float32),
        pltpu.VMEM((_GB, _D), jnp.float32),
        pltpu.VMEM((_OWN + 8, _D), jnp.float32),
        pltpu.VMEM((128,), jnp.int32),
        pltpu.SemaphoreType.DMA,
        pltpu.SemaphoreType.DMA,
    ]
    if compute_deg:
        out_types.append(jax.ShapeDtypeStruct((_NPAD * 16,), jnp.float32))
        scratch.append(pltpu.VMEM(((_OWN + 8) * 16,), jnp.float32))

    @functools.partial(
        pl.kernel,
        out_type=out_types,
        mesh=_mesh(),
        compiler_params=_SC_PARAMS,
        scratch_types=scratch,
    )
    def agg(table_hbm, q_hbm, cnt_hbm, zd_hbm, z16_hbm, *refs):
        if compute_deg:
            (out_hbm, deg_hbm, sq_v, gidx_v, lidx_v, rows0, rows1, acc_v, cv,
             sem0, sem1, deg_v) = refs
        else:
            (out_hbm, sq_v, gidx_v, lidx_v, rows0, rows1, acc_v, cv,
             sem0, sem1) = refs
        rows = (rows0, rows1)
        sems = (sem0, sem1)
        c = lax.axis_index("c")
        s = lax.axis_index("s")
        w = c * 16 + s

        pltpu.sync_copy(zd_hbm, acc_v)
        if compute_deg:
            pltpu.sync_copy(z16_hbm, deg_v)
        pltpu.sync_copy(cnt_hbm.at[pl.ds(pl.multiple_of(w * 128, 128), 128)],
                        cv)
        qn = cv[pl.ds(0, 16)][0]
        one = jnp.ones((16,), jnp.float32)

        def prep_fire(k):
            for j in range(_GB // 16):
                v = sq_v[pl.ds(k * _GB + j * 16, 16)]
                gidx_v[k, pl.ds(j * 16, 16)] = v >> 9
                lidx_v[k, pl.ds(j * 16, 16)] = v & 511
            pltpu.async_copy(table_hbm.at[gidx_v.at[k]], rows[k % 2],
                             sems[k % 2])

        def acc_block(k):
            pltpu.make_async_copy(table_hbm.at[gidx_v.at[k]], rows[k % 2],
                                  sems[k % 2]).wait()

            def row_group(g, carry2):
                lv = lidx_v[k, pl.ds(g * 16, 16)]
                for i in range(16):
                    ld = lv[i]
                    r = g * 16 + i
                    vals = [rows[k % 2][r, pl.ds(j * 16, 16)]
                            for j in range(_D // 16)]
                    for j in range(_D // 16):
                        plsc.addupdate(acc_v.at[ld, pl.ds(j * 16, 16)],
                                       vals[j])
                    if compute_deg:
                        plsc.addupdate(deg_v.at[pl.ds(ld * 16, 16)], one)
                return carry2

            lax.fori_loop(0, _GB // 16, row_group, 0)

        def superchunk(base, nblk):
            o = pl.multiple_of(w * _QCAP + base, 128)
            pltpu.sync_copy(q_hbm.at[pl.ds(o, _SQ)], sq_v)
            for k in range(_NBLK):
                if isinstance(nblk, int):
                    prep_fire(k)
                    if k >= 1:
                        acc_block(k - 1)
                else:
                    @pl.when(k < nblk)
                    def _():
                        prep_fire(k)
                    if k >= 1:
                        @pl.when(k - 1 < nblk)
                        def _():
                            acc_block(k - 1)
            if isinstance(nblk, int):
                acc_block(_NBLK - 1)
            else:
                @pl.when(nblk >= _NBLK)
                def _():
                    acc_block(_NBLK - 1)

        nsq = qn // _SQ

        def full(sc, carry):
            superchunk(sc * _SQ, _NBLK)
            return carry

        lax.fori_loop(0, nsq, full, 0)
        superchunk(nsq * _SQ, (qn - nsq * _SQ) // _GB)

        pltpu.sync_copy(acc_v.at[pl.ds(0, _OWN)],
                        out_hbm.at[pl.ds(pl.multiple_of(w * _OWN, 8), _OWN)])
        if compute_deg:
            pltpu.sync_copy(deg_v.at[pl.ds(0, _OWN * 16)],
                            deg_hbm.at[pl.ds(pl.multiple_of(w * _OWN * 16,
                                                            128),
                                             _OWN * 16)])

    return agg


_aggregate_deg = _make_aggregate(True)
_aggregate_nodeg = _make_aggregate(False)

_BR = 1000


def _sage_layer_tc(aggsum, deg16, h_in, Wl, Wr, b, relu: bool):
    def body(agg_ref, deg_ref, h_ref, wl_ref, wr_ref, b_ref, o_ref):
        deg = jnp.maximum(deg_ref[:, 0:1], 1.0)
        agg = agg_ref[...] / deg
        o = (jnp.dot(agg, wl_ref[...], preferred_element_type=jnp.float32)
             + jnp.dot(h_ref[...], wr_ref[...],
                       preferred_element_type=jnp.float32)
             + b_ref[...])
        if relu:
            o = jnp.maximum(o, 0.0)
        o_ref[...] = o

    return pl.pallas_call(
        body,
        grid=(_N // _BR,),
        in_specs=[
            pl.BlockSpec((_BR, _D), lambda i: (i, 0)),
            pl.BlockSpec((_BR, 16), lambda i: (i, 0)),
            pl.BlockSpec((_BR, _D), lambda i: (i, 0)),
            pl.BlockSpec((_D, _H), lambda i: (0, 0)),
            pl.BlockSpec((_D, _H), lambda i: (0, 0)),
            pl.BlockSpec((1, _H), lambda i: (0, 0)),
        ],
        out_specs=pl.BlockSpec((_BR, _H), lambda i: (i, 0)),
        out_shape=jax.ShapeDtypeStruct((_N, _H), jnp.float32),
    )(aggsum, deg16, h_in, Wl, Wr, b.reshape(1, _H))


def _final_tc(aggsum, deg16, h_in, Wl, Wr, b, Wfc, bfc):
    def body(agg_ref, deg_ref, h_ref, wl_ref, wr_ref, b_ref, wfc_ref,
             bfc_ref, o_ref):
        deg = jnp.maximum(deg_ref[:, 0:1], 1.0)
        agg = agg_ref[...] / deg
        h2 = (jnp.dot(agg, wl_ref[...], preferred_element_type=jnp.float32)
              + jnp.dot(h_ref[...], wr_ref[...],
                        preferred_element_type=jnp.float32)
              + b_ref[...])
        z = (jnp.dot(h2, wfc_ref[...], preferred_element_type=jnp.float32)
             + bfc_ref[...])
        m = jnp.max(z, axis=-1, keepdims=True)
        e = jnp.exp(z - m)
        o_ref[...] = z - m - jnp.log(jnp.sum(e, axis=-1, keepdims=True))

    return pl.pallas_call(
        body,
        grid=(_N // _BR,),
        in_specs=[
            pl.BlockSpec((_BR, _D), lambda i: (i, 0)),
            pl.BlockSpec((_BR, 16), lambda i: (i, 0)),
            pl.BlockSpec((_BR, _H), lambda i: (i, 0)),
            pl.BlockSpec((_H, _H), lambda i: (0, 0)),
            pl.BlockSpec((_H, _H), lambda i: (0, 0)),
            pl.BlockSpec((1, _H), lambda i: (0, 0)),
            pl.BlockSpec((_H, _C), lambda i: (0, 0)),
            pl.BlockSpec((1, _C), lambda i: (0, 0)),
        ],
        out_specs=pl.BlockSpec((_BR, _C), lambda i: (i, 0)),
        out_shape=jax.ShapeDtypeStruct((_N, _C), jnp.float32),
    )(aggsum, deg16, h_in, Wl, Wr, b.reshape(1, _H), Wfc, bfc.reshape(1, _C))


def kernel(x, edge_index, W_l0, W_r0, b0, W_l1, W_r1, b1, W_fc, b_fc):
    src = edge_index[0]
    dst = edge_index[1]
    zd = jnp.zeros((_OWN + 8, _D), jnp.float32)
    z16 = jnp.zeros(((_OWN + 8) * 16,), jnp.float32)

    q, counts = _scan(src, dst)
    agg0, deg = _aggregate_deg(x, q, counts, zd, z16)
    deg = deg.reshape(_NPAD, 16)
    h = _sage_layer_tc(agg0, deg, x, W_l0, W_r0, b0, relu=True)
    (agg1,) = _aggregate_nodeg(h, q, counts, zd, z16)
    return _final_tc(agg1, deg, h, W_l1, W_r1, b1, W_fc, b_fc)


# speedup vs baseline: 3.1413x; 1.1914x over previous
import functools

import jax
import jax.numpy as jnp
from jax import lax
from jax.experimental import pallas as pl
from jax.experimental.pallas import tpu as pltpu
from jax.experimental.pallas import tpu_sc as plsc

_N = 10000
_E = 160000
_D = 256
_H = 256
_C = 64

_NT = 32
_OWN = 320
_NPAD = _NT * _OWN
_DUMMY = _OWN
_KS = 1280
_NCHUNK = _E // _KS
_K = 128
_GB = 64
_SQ = 256
_NBLK = _SQ // _GB
_QCAP = 626 * _SQ
_QV = _KS + 256

_SC_PARAMS = pltpu.CompilerParams(needs_layout_passes=False)


def _mesh():
    return plsc.VectorSubcoreMesh(core_axis_name="c", subcore_axis_name="s")


@functools.partial(
    pl.kernel,
    out_type=[jax.ShapeDtypeStruct((_NT * _QCAP,), jnp.int32),
              jax.ShapeDtypeStruct((_NT * 128,), jnp.int32)],
    mesh=_mesh(),
    compiler_params=_SC_PARAMS,
    scratch_types=[
        pltpu.VMEM((_KS,), jnp.int32),
        pltpu.VMEM((_KS,), jnp.int32),
        pltpu.VMEM((_KS,), jnp.int32),
        pltpu.VMEM((_KS,), jnp.int32),
        pltpu.VMEM((_QV,), jnp.int32),
        pltpu.VMEM((128,), jnp.int32),
        pltpu.SemaphoreType.DMA,
        pltpu.SemaphoreType.DMA,
    ],
)
def _scan(src_hbm, dst_hbm, q_hbm, cnt_hbm, sstage0, dstage0, sstage1,
          dstage1, q_v, cv, sem0, sem1):
    c = lax.axis_index("c")
    s = lax.axis_index("s")
    w = c * 16 + s
    lo = w * _OWN
    stages = ((sstage0, dstage0, sem0), (sstage1, dstage1, sem1))

    def fire(t, p):
        sb, db, sem = stages[p]
        e0 = pl.multiple_of(t * _KS, 128)
        pltpu.async_copy(src_hbm.at[pl.ds(e0, _KS)], sb, sem)
        pltpu.async_copy(dst_hbm.at[pl.ds(e0, _KS)], db, sem)

    def wait(p):
        sb, db, sem = stages[p]
        pltpu.make_async_copy(src_hbm.at[pl.ds(0, _KS)], sb, sem).wait()
        pltpu.make_async_copy(dst_hbm.at[pl.ds(0, _KS)], db, sem).wait()

    def scan_chunk(p, cnt, qtot):
        sb, db, _ = stages[p]

        def vec(i, cnt):
            sv = sb[pl.ds(i * 16, 16)]
            dv = db[pl.ds(i * 16, 16)]
            rel = dv - lo
            own = rel.astype(jnp.uint32) < jnp.uint32(_OWN)
            pk = (sv << 9) | jnp.where(own, rel, _DUMMY)
            key = jnp.where(own, 0, 1)
            _, vv = plsc.sort_key_val(key, pk)
            q_v[pl.ds(cnt, 16)] = vv
            return cnt + plsc.all_reduce_population_count(own)[0]

        cnt = lax.fori_loop(0, _KS // 16, vec, cnt)

        nb = cnt // _K

        def flush(b, carry2):
            o = pl.multiple_of(w * _QCAP + qtot + b * _K, 128)
            pltpu.sync_copy(q_v.at[pl.ds(b * _K, _K)], q_hbm.at[pl.ds(o, _K)])
            return carry2

        lax.fori_loop(0, nb, flush, 0)
        for j in range(_K // 16):
            q_v[pl.ds(j * 16, 16)] = q_v[pl.ds(nb * _K + j * 16, 16)]
        return cnt - nb * _K, qtot + nb * _K

    fire(0, 0)

    def pair(tp, carry):
        cnt, qtot = carry
        fire(2 * tp + 1, 1)
        wait(0)
        cnt, qtot = scan_chunk(0, cnt, qtot)

        @pl.when(2 * tp + 2 < _NCHUNK)
        def _():
            fire(2 * tp + 2, 0)

        wait(1)
        return scan_chunk(1, cnt, qtot)

    cnt, qtot = lax.fori_loop(0, _NCHUNK // 2, pair, (0, 0))
    wait(0)
    cnt, qtot = scan_chunk(0, cnt, qtot)

    pad = jnp.full((16,), _DUMMY, jnp.int32)
    for j in range(_K // 16):
        q_v[pl.ds(cnt + j * 16, 16)] = pad
    o = pl.multiple_of(w * _QCAP + qtot, 128)
    pltpu.sync_copy(q_v.at[pl.ds(0, _K)], q_hbm.at[pl.ds(o, _K)])
    qtot = qtot + _K

    for j in range(8):
        cv[pl.ds(j * 16, 16)] = jnp.zeros((16,), jnp.int32) + qtot
    pltpu.sync_copy(cv, cnt_hbm.at[pl.ds(pl.multiple_of(w * 128, 128), 128)])


def _make_aggregate(compute_deg: bool):
    out_types = [jax.ShapeDtypeStruct((_NPAD, _D), jnp.float32)]
    scratch = [
        pltpu.VMEM((_SQ,), jnp.int32),
        pltpu.VMEM((_NBLK, _GB), jnp.int32),
        pltpu.VMEM((_NBLK, _GB + 16), jnp.int32),
        pltpu.VMEM((_GB, _D), jnp.float32),
        pltpu.VMEM((_GB, _D), jnp.float32),
        pltpu.VMEM((_OWN + 8, _D), jnp.float32),
        pltpu.VMEM((128,), jnp.int32),
        pltpu.SemaphoreType.DMA,
        pltpu.SemaphoreType.DMA,
    ]
    if compute_deg:
        out_types.append(jax.ShapeDtypeStruct((_NPAD * 16,), jnp.float32))
        scratch.append(pltpu.VMEM(((_OWN + 8) * 16,), jnp.float32))

    @functools.partial(
        pl.kernel,
        out_type=out_types,
        mesh=_mesh(),
        compiler_params=_SC_PARAMS,
        scratch_types=scratch,
    )
    def agg(table_hbm, q_hbm, cnt_hbm, zd_hbm, z16_hbm, *refs):
        if compute_deg:
            (out_hbm, deg_hbm, sq_v, gidx_v, lidx_v, rows0, rows1, acc_v, cv,
             sem0, sem1, deg_v) = refs
        else:
            (out_hbm, sq_v, gidx_v, lidx_v, rows0, rows1, acc_v, cv,
             sem0, sem1) = refs
        rows = (rows0, rows1)
        sems = (sem0, sem1)
        c = lax.axis_index("c")
        s = lax.axis_index("s")
        w = c * 16 + s

        pltpu.sync_copy(zd_hbm, acc_v)
        if compute_deg:
            pltpu.sync_copy(z16_hbm, deg_v)
        pltpu.sync_copy(cnt_hbm.at[pl.ds(pl.multiple_of(w * 128, 128), 128)],
                        cv)
        qn = cv[pl.ds(0, 16)][0]
        one = jnp.ones((16,), jnp.float32)

        def prep_fire(k):
            for j in range(_GB // 16):
                v = sq_v[pl.ds(k * _GB + j * 16, 16)]
                gidx_v[k, pl.ds(j * 16, 16)] = v >> 9
                lidx_v[k, pl.ds(j * 16, 16)] = v & 511
            pltpu.async_copy(table_hbm.at[gidx_v.at[k]], rows[k % 2],
                             sems[k % 2])

        def acc_block(k):
            pltpu.make_async_copy(table_hbm.at[gidx_v.at[k]], rows[k % 2],
                                  sems[k % 2]).wait()

            def row_group(g, carry2):
                lv = lidx_v[k, pl.ds(g * 16, 16)]
                for i in range(16):
                    ld = lv[i]
                    r = g * 16 + i
                    vals = [rows[k % 2][r, pl.ds(j * 16, 16)]
                            for j in range(_D // 16)]
                    for j in range(_D // 16):
                        plsc.addupdate(acc_v.at[ld, pl.ds(j * 16, 16)],
                                       vals[j])
                    if compute_deg:
                        plsc.addupdate(deg_v.at[pl.ds(ld * 16, 16)], one)
                return carry2

            lax.fori_loop(0, _GB // 16, row_group, 0)

        def superchunk(base, nblk):
            o = pl.multiple_of(w * _QCAP + base, 128)
            pltpu.sync_copy(q_hbm.at[pl.ds(o, _SQ)], sq_v)
            for k in range(_NBLK):
                if isinstance(nblk, int):
                    prep_fire(k)
                    if k >= 1:
                        acc_block(k - 1)
                else:
                    @pl.when(k < nblk)
                    def _():
                        prep_fire(k)
                    if k >= 1:
                        @pl.when(k - 1 < nblk)
                        def _():
                            acc_block(k - 1)
            if isinstance(nblk, int):
                acc_block(_NBLK - 1)
            else:
                @pl.when(nblk >= _NBLK)
                def _():
                    acc_block(_NBLK - 1)

        nsq = qn // _SQ

        def full(sc, carry):
            superchunk(sc * _SQ, _NBLK)
            return carry

        lax.fori_loop(0, nsq, full, 0)
        superchunk(nsq * _SQ, (qn - nsq * _SQ) // _GB)

        pltpu.sync_copy(acc_v.at[pl.ds(0, _OWN)],
                        out_hbm.at[pl.ds(pl.multiple_of(w * _OWN, 8), _OWN)])
        if compute_deg:
            pltpu.sync_copy(deg_v.at[pl.ds(0, _OWN * 16)],
                            deg_hbm.at[pl.ds(pl.multiple_of(w * _OWN * 16,
                                                            128),
                                             _OWN * 16)])

    return agg


_aggregate_deg = _make_aggregate(True)
_aggregate_nodeg = _make_aggregate(False)

_BR = 1000


def _sage_layer_tc(aggsum, deg16, h_in, Wl, Wr, b, relu: bool):
    def body(agg_ref, deg_ref, h_ref, wl_ref, wr_ref, b_ref, o_ref):
        deg = jnp.maximum(deg_ref[:, 0:1], 1.0)
        agg = agg_ref[...] / deg
        o = (jnp.dot(agg, wl_ref[...], preferred_element_type=jnp.float32)
             + jnp.dot(h_ref[...], wr_ref[...],
                       preferred_element_type=jnp.float32)
             + b_ref[...])
        if relu:
            o = jnp.maximum(o, 0.0)
        o_ref[...] = o

    return pl.pallas_call(
        body,
        grid=(_N // _BR,),
        in_specs=[
            pl.BlockSpec((_BR, _D), lambda i: (i, 0)),
            pl.BlockSpec((_BR, 16), lambda i: (i, 0)),
            pl.BlockSpec((_BR, _D), lambda i: (i, 0)),
            pl.BlockSpec((_D, _H), lambda i: (0, 0)),
            pl.BlockSpec((_D, _H), lambda i: (0, 0)),
            pl.BlockSpec((1, _H), lambda i: (0, 0)),
        ],
        out_specs=pl.BlockSpec((_BR, _H), lambda i: (i, 0)),
        out_shape=jax.ShapeDtypeStruct((_N, _H), jnp.float32),
    )(aggsum, deg16, h_in, Wl, Wr, b.reshape(1, _H))


def _final_tc(aggsum, deg16, h_in, Wl, Wr, b, Wfc, bfc):
    def body(agg_ref, deg_ref, h_ref, wl_ref, wr_ref, b_ref, wfc_ref,
             bfc_ref, o_ref):
        deg = jnp.maximum(deg_ref[:, 0:1], 1.0)
        agg = agg_ref[...] / deg
        h2 = (jnp.dot(agg, wl_ref[...], preferred_element_type=jnp.float32)
              + jnp.dot(h_ref[...], wr_ref[...],
                        preferred_element_type=jnp.float32)
              + b_ref[...])
        z = (jnp.dot(h2, wfc_ref[...], preferred_element_type=jnp.float32)
             + bfc_ref[...])
        m = jnp.max(z, axis=-1, keepdims=True)
        e = jnp.exp(z - m)
        o_ref[...] = z - m - jnp.log(jnp.sum(e, axis=-1, keepdims=True))

    return pl.pallas_call(
        body,
        grid=(_N // _BR,),
        in_specs=[
            pl.BlockSpec((_BR, _D), lambda i: (i, 0)),
            pl.BlockSpec((_BR, 16), lambda i: (i, 0)),
            pl.BlockSpec((_BR, _H), lambda i: (i, 0)),
            pl.BlockSpec((_H, _H), lambda i: (0, 0)),
            pl.BlockSpec((_H, _H), lambda i: (0, 0)),
            pl.BlockSpec((1, _H), lambda i: (0, 0)),
            pl.BlockSpec((_H, _C), lambda i: (0, 0)),
            pl.BlockSpec((1, _C), lambda i: (0, 0)),
        ],
        out_specs=pl.BlockSpec((_BR, _C), lambda i: (i, 0)),
        out_shape=jax.ShapeDtypeStruct((_N, _C), jnp.float32),
    )(aggsum, deg16, h_in, Wl, Wr, b.reshape(1, _H), Wfc, bfc.reshape(1, _C))


def kernel(x, edge_index, W_l0, W_r0, b0, W_l1, W_r1, b1, W_fc, b_fc):
    src = edge_index[0]
    dst = edge_index[1]
    zd = jnp.zeros((_OWN + 8, _D), jnp.float32)
    z16 = jnp.zeros(((_OWN + 8) * 16,), jnp.float32)

    q, counts = _scan(src, dst)
    agg0, deg = _aggregate_deg(x, q, counts, zd, z16)
    deg = deg.reshape(_NPAD, 16)
    h = _sage_layer_tc(agg0, deg, x, W_l0, W_r0, b0, relu=True)
    (agg1,) = _aggregate_nodeg(h, q, counts, zd, z16)
    return _final_tc(agg1, deg, h, W_l1, W_r1, b1, W_fc, b_fc)
